# baseline (speedup 1.0000x reference)
import functools

import jax
import jax.numpy as jnp
from jax import lax
from jax.experimental import pallas as pl
from jax.experimental.pallas import tpu as pltpu
from jax.experimental.pallas import tpu_sc as plsc

VOCAB = 1000000
D = 32
B = 16384
H = 200

CV = 2048
NBLK = -(-VOCAB // CV)
VP = NBLK * CV

NC = 2
NS = 16
NW = NC * NS
RPW = B // NW
IC = 32
NGRP = RPW // IC
G0 = 128
G1 = H - G0
U = 8
NACC = 4
NBUF = 4

_mesh = plsc.VectorSubcoreMesh(
    core_axis_name="c", subcore_axis_name="s", num_cores=NC, num_subcores=NS
)


def _repack_body(in_ref, out_ref):
    eye = jnp.eye(D, dtype=jnp.float32)
    tt = jax.lax.dot_general(
        in_ref[...], eye, (((0,), (0,)), ((), ())),
        precision=jax.lax.Precision.HIGHEST,
        preferred_element_type=jnp.float32,
    )
    r3 = tt.reshape(CV // 4, 4, D)
    for a in range(4):
        out_ref[:, D * a:D * (a + 1)] = r3[:, a, :]


def _tc_repack(wt):
    return pl.pallas_call(
        _repack_body,
        grid=(NBLK,),
        in_specs=[pl.BlockSpec((D, CV), lambda c: (0, c))],
        out_specs=pl.BlockSpec((CV // 4, 128), lambda c: (c, 0)),
        out_shape=jax.ShapeDtypeStruct((VP // 4, 128), jnp.float32),
    )(wt)


@functools.partial(
    pl.kernel,
    out_type=jax.ShapeDtypeStruct((B, D), jnp.float32),
    mesh=_mesh,
    compiler_params=pltpu.CompilerParams(use_tc_tiling_on_sc=False),
    scratch_types=[
        pltpu.VMEM((IC, H), jnp.int32),
        pltpu.VMEM((NBUF, H, D), jnp.float32),
        pltpu.VMEM((IC, D), jnp.float32),
    ]
    + [pltpu.SemaphoreType.DMA for _ in range(NBUF)],
)
def _encode(x_hbm, w_hbm, out_hbm, idx_v, rows_v, out_v, *sems):
    wid = lax.axis_index("s") * NC + lax.axis_index("c")
    base_row = wid * RPW
    inv_h = jnp.float32(1.0 / H)

    def copies(r, b):
        row_idx = idx_v.at[r]
        cp0 = pltpu.make_async_copy(
            w_hbm.at[row_idx.at[pl.ds(0, G0)]],
            rows_v.at[b].at[pl.ds(0, G0)],
            sems[b],
        )
        cp1 = pltpu.make_async_copy(
            w_hbm.at[row_idx.at[pl.ds(G0, G1)]],
            rows_v.at[b].at[pl.ds(G0, G1)],
            sems[b],
        )
        return cp0, cp1

    def fire(r, b):
        for cp in copies(r, b):
            cp.start()

    def drain(r, b):
        for cp in copies(r, b):
            cp.wait()

    def accumulate(b):
        def acc_body(it, carry):
            a = list(carry)
            base = it * U
            for u in range(U):
                j = base + u
                k = u % NACC
                a[2 * k] = a[2 * k] + rows_v[b, j, pl.ds(0, 16)]
                a[2 * k + 1] = a[2 * k + 1] + rows_v[b, j, pl.ds(16, 16)]
            return tuple(a)

        zeros = tuple(jnp.zeros((16,), jnp.float32) for _ in range(2 * NACC))
        a = lax.fori_loop(0, H // U, acc_body, zeros)
        lo = (a[0] + a[2]) + (a[4] + a[6])
        hi = (a[1] + a[3]) + (a[5] + a[7])
        return lo * inv_h, hi * inv_h

    def group_body(g, _):
        grp_row = base_row + g * IC
        pltpu.sync_copy(x_hbm.at[pl.ds(grp_row, IC)], idx_v)
        for b in range(NBUF):
            fire(b, b)

        def ring_body(rr, _):
            for b in range(NBUF):
                r = rr * NBUF + b
                drain(r, b)

                @pl.when(rr < IC // NBUF - 1)
                def _():
                    fire(r + NBUF, b)

                lo, hi = accumulate(b)
                out_v[r, pl.ds(0, 16)] = lo
                out_v[r, pl.ds(16, 16)] = hi
            return 0

        lax.fori_loop(0, IC // NBUF, ring_body, 0)
        pltpu.sync_copy(out_v, out_hbm.at[pl.ds(grp_row, IC)])
        return 0

    lax.fori_loop(0, NGRP, group_body, 0)


def kernel(x, W):
    w_lin = _tc_repack(W.T).reshape(VP, D)
    return _encode(x, w_lin)


# speedup vs baseline: 1.3812x; 1.3812x over previous
import functools

import jax
import jax.numpy as jnp
from jax import lax
from jax.experimental import pallas as pl
from jax.experimental.pallas import tpu as pltpu
from jax.experimental.pallas import tpu_sc as plsc

VOCAB = 1000000
D = 32
B = 16384
H = 200

CV = 2048
NBLK = -(-VOCAB // CV)
VP = NBLK * CV

NC = 2
NS = 16
NW = NC * NS
RPW = B // NW
IC = 32
NGRP = RPW // IC
G0 = 128
G1 = H - G0
U = 8
NACC = 4
NBUF = 4

_mesh = plsc.VectorSubcoreMesh(
    core_axis_name="c", subcore_axis_name="s", num_cores=NC, num_subcores=NS
)


def _repack_body(in_ref, out_ref):
    tt = jnp.swapaxes(in_ref[...], 0, 1)
    r3 = tt.reshape(CV // 4, 4, D)
    for a in range(4):
        out_ref[:, D * a:D * (a + 1)] = r3[:, a, :]


def _tc_repack(wt):
    return pl.pallas_call(
        _repack_body,
        grid=(NBLK,),
        in_specs=[pl.BlockSpec((D, CV), lambda c: (0, c))],
        out_specs=pl.BlockSpec((CV // 4, 128), lambda c: (c, 0)),
        out_shape=jax.ShapeDtypeStruct((VP // 4, 128), jnp.float32),
    )(wt)


@functools.partial(
    pl.kernel,
    out_type=jax.ShapeDtypeStruct((B, D), jnp.float32),
    mesh=_mesh,
    compiler_params=pltpu.CompilerParams(use_tc_tiling_on_sc=False),
    scratch_types=[
        pltpu.VMEM((IC, H), jnp.int32),
        pltpu.VMEM((NBUF, H, D), jnp.float32),
        pltpu.VMEM((IC, D), jnp.float32),
    ]
    + [pltpu.SemaphoreType.DMA for _ in range(NBUF)],
)
def _encode(x_hbm, w_hbm, out_hbm, idx_v, rows_v, out_v, *sems):
    wid = lax.axis_index("s") * NC + lax.axis_index("c")
    base_row = wid * RPW
    inv_h = jnp.float32(1.0 / H)

    def copies(r, b):
        row_idx = idx_v.at[r]
        cp0 = pltpu.make_async_copy(
            w_hbm.at[row_idx.at[pl.ds(0, G0)]],
            rows_v.at[b].at[pl.ds(0, G0)],
            sems[b],
        )
        cp1 = pltpu.make_async_copy(
            w_hbm.at[row_idx.at[pl.ds(G0, G1)]],
            rows_v.at[b].at[pl.ds(G0, G1)],
            sems[b],
        )
        return cp0, cp1

    def fire(r, b):
        for cp in copies(r, b):
            cp.start()

    def drain(r, b):
        for cp in copies(r, b):
            cp.wait()

    def accumulate(b):
        def acc_body(it, carry):
            a = list(carry)
            base = it * U
            for u in range(U):
                j = base + u
                k = u % NACC
                a[2 * k] = a[2 * k] + rows_v[b, j, pl.ds(0, 16)]
                a[2 * k + 1] = a[2 * k + 1] + rows_v[b, j, pl.ds(16, 16)]
            return tuple(a)

        zeros = tuple(jnp.zeros((16,), jnp.float32) for _ in range(2 * NACC))
        a = lax.fori_loop(0, H // U, acc_body, zeros)
        lo = (a[0] + a[2]) + (a[4] + a[6])
        hi = (a[1] + a[3]) + (a[5] + a[7])
        return lo * inv_h, hi * inv_h

    def group_body(g, _):
        grp_row = base_row + g * IC
        pltpu.sync_copy(x_hbm.at[pl.ds(grp_row, IC)], idx_v)
        for b in range(NBUF):
            fire(b, b)

        def ring_body(rr, _):
            for b in range(NBUF):
                r = rr * NBUF + b
                drain(r, b)

                @pl.when(rr < IC // NBUF - 1)
                def _():
                    fire(r + NBUF, b)

                lo, hi = accumulate(b)
                out_v[r, pl.ds(0, 16)] = lo
                out_v[r, pl.ds(16, 16)] = hi
            return 0

        lax.fori_loop(0, IC // NBUF, ring_body, 0)
        pltpu.sync_copy(out_v, out_hbm.at[pl.ds(grp_row, IC)])
        return 0

    lax.fori_loop(0, NGRP, group_body, 0)


def kernel(x, W):
    w_lin = _tc_repack(W.T).reshape(VP, D)
    return _encode(x, w_lin)


# speedup vs baseline: 1.5969x; 1.1561x over previous
import functools

import jax
import jax.numpy as jnp
from jax import lax
from jax.experimental import pallas as pl
from jax.experimental.pallas import tpu as pltpu
from jax.experimental.pallas import tpu_sc as plsc

VOCAB = 1000000
D = 32
B = 16384
H = 200

CV = 4096
NBLK = -(-VOCAB // CV)
VP = NBLK * CV

NC = 2
NS = 16
NW = NC * NS
RPW = B // NW
IC = 32
NGRP = RPW // IC
G0 = 128
G1 = H - G0
U = 8
NACC = 4
NBUF = 4

_mesh = plsc.VectorSubcoreMesh(
    core_axis_name="c", subcore_axis_name="s", num_cores=NC, num_subcores=NS
)


def _repack_body(in_ref, out_ref):
    tt = jnp.swapaxes(in_ref[...], 0, 1)
    r3 = tt.reshape(CV // 4, 4, D)
    for a in range(4):
        out_ref[:, D * a:D * (a + 1)] = r3[:, a, :]


def _tc_repack(wt):
    return pl.pallas_call(
        _repack_body,
        grid=(NBLK,),
        in_specs=[pl.BlockSpec((D, CV), lambda c: (0, c))],
        out_specs=pl.BlockSpec((CV // 4, 128), lambda c: (c, 0)),
        out_shape=jax.ShapeDtypeStruct((VP // 4, 128), jnp.float32),
    )(wt)


@functools.partial(
    pl.kernel,
    out_type=jax.ShapeDtypeStruct((B, D), jnp.float32),
    mesh=_mesh,
    compiler_params=pltpu.CompilerParams(use_tc_tiling_on_sc=False),
    scratch_types=[
        pltpu.VMEM((IC, H), jnp.int32),
        pltpu.VMEM((NBUF, H, D), jnp.float32),
        pltpu.VMEM((IC, D), jnp.float32),
    ]
    + [pltpu.SemaphoreType.DMA for _ in range(NBUF)],
)
def _encode(x_hbm, w_hbm, out_hbm, idx_v, rows_v, out_v, *sems):
    wid = lax.axis_index("s") * NC + lax.axis_index("c")
    base_row = wid * RPW
    inv_h = jnp.float32(1.0 / H)

    def copies(r, b):
        row_idx = idx_v.at[r]
        cp0 = pltpu.make_async_copy(
            w_hbm.at[row_idx.at[pl.ds(0, G0)]],
            rows_v.at[b].at[pl.ds(0, G0)],
            sems[b],
        )
        cp1 = pltpu.make_async_copy(
            w_hbm.at[row_idx.at[pl.ds(G0, G1)]],
            rows_v.at[b].at[pl.ds(G0, G1)],
            sems[b],
        )
        return cp0, cp1

    def fire(r, b):
        for cp in copies(r, b):
            cp.start()

    def drain(r, b):
        for cp in copies(r, b):
            cp.wait()

    def accumulate(b):
        def acc_body(it, carry):
            a = list(carry)
            base = it * U
            for u in range(U):
                j = base + u
                k = u % NACC
                a[2 * k] = a[2 * k] + rows_v[b, j, pl.ds(0, 16)]
                a[2 * k + 1] = a[2 * k + 1] + rows_v[b, j, pl.ds(16, 16)]
            return tuple(a)

        zeros = tuple(jnp.zeros((16,), jnp.float32) for _ in range(2 * NACC))
        a = lax.fori_loop(0, H // U, acc_body, zeros)
        lo = (a[0] + a[2]) + (a[4] + a[6])
        hi = (a[1] + a[3]) + (a[5] + a[7])
        return lo * inv_h, hi * inv_h

    def group_body(g, _):
        grp_row = base_row + g * IC
        pltpu.sync_copy(x_hbm.at[pl.ds(grp_row, IC)], idx_v)
        for b in range(NBUF):
            fire(b, b)

        def ring_body(rr, _):
            for b in range(NBUF):
                r = rr * NBUF + b
                drain(r, b)

                @pl.when(rr < IC // NBUF - 1)
                def _():
                    fire(r + NBUF, b)

                lo, hi = accumulate(b)
                out_v[r, pl.ds(0, 16)] = lo
                out_v[r, pl.ds(16, 16)] = hi
            return 0

        lax.fori_loop(0, IC // NBUF, ring_body, 0)
        pltpu.sync_copy(out_v, out_hbm.at[pl.ds(grp_row, IC)])
        return 0

    lax.fori_loop(0, NGRP, group_body, 0)


def kernel(x, W):
    w_lin = _tc_repack(W.T).reshape(VP, D)
    return _encode(x, w_lin)


# speedup vs baseline: 1.6614x; 1.0404x over previous
import functools

import jax
import jax.numpy as jnp
from jax import lax
from jax.experimental import pallas as pl
from jax.experimental.pallas import tpu as pltpu
from jax.experimental.pallas import tpu_sc as plsc

VOCAB = 1000000
D = 32
B = 16384
H = 200

CV = 8192
NBLK = -(-VOCAB // CV)
VP = NBLK * CV

NC = 2
NS = 16
NW = NC * NS
RPW = B // NW
IC = 32
NGRP = RPW // IC
G0 = 128
G1 = H - G0
U = 8
NACC = 4
NBUF = 4

_mesh = plsc.VectorSubcoreMesh(
    core_axis_name="c", subcore_axis_name="s", num_cores=NC, num_subcores=NS
)


def _repack_body(in_ref, out_ref):
    tt = jnp.swapaxes(in_ref[...], 0, 1)
    r3 = tt.reshape(CV // 4, 4, D)
    for a in range(4):
        out_ref[:, D * a:D * (a + 1)] = r3[:, a, :]


def _tc_repack(wt):
    return pl.pallas_call(
        _repack_body,
        grid=(NBLK,),
        in_specs=[pl.BlockSpec((D, CV), lambda c: (0, c))],
        out_specs=pl.BlockSpec((CV // 4, 128), lambda c: (c, 0)),
        out_shape=jax.ShapeDtypeStruct((VP // 4, 128), jnp.float32),
    )(wt)


@functools.partial(
    pl.kernel,
    out_type=jax.ShapeDtypeStruct((B, D), jnp.float32),
    mesh=_mesh,
    compiler_params=pltpu.CompilerParams(use_tc_tiling_on_sc=False),
    scratch_types=[
        pltpu.VMEM((IC, H), jnp.int32),
        pltpu.VMEM((NBUF, H, D), jnp.float32),
        pltpu.VMEM((IC, D), jnp.float32),
    ]
    + [pltpu.SemaphoreType.DMA for _ in range(NBUF)],
)
def _encode(x_hbm, w_hbm, out_hbm, idx_v, rows_v, out_v, *sems):
    wid = lax.axis_index("s") * NC + lax.axis_index("c")
    base_row = wid * RPW
    inv_h = jnp.float32(1.0 / H)

    def copies(r, b):
        row_idx = idx_v.at[r]
        cp0 = pltpu.make_async_copy(
            w_hbm.at[row_idx.at[pl.ds(0, G0)]],
            rows_v.at[b].at[pl.ds(0, G0)],
            sems[b],
        )
        cp1 = pltpu.make_async_copy(
            w_hbm.at[row_idx.at[pl.ds(G0, G1)]],
            rows_v.at[b].at[pl.ds(G0, G1)],
            sems[b],
        )
        return cp0, cp1

    def fire(r, b):
        for cp in copies(r, b):
            cp.start()

    def drain(r, b):
        for cp in copies(r, b):
            cp.wait()

    def accumulate(b):
        def acc_body(it, carry):
            a = list(carry)
            base = it * U
            for u in range(U):
                j = base + u
                k = u % NACC
                a[2 * k] = a[2 * k] + rows_v[b, j, pl.ds(0, 16)]
                a[2 * k + 1] = a[2 * k + 1] + rows_v[b, j, pl.ds(16, 16)]
            return tuple(a)

        zeros = tuple(jnp.zeros((16,), jnp.float32) for _ in range(2 * NACC))
        a = lax.fori_loop(0, H // U, acc_body, zeros)
        lo = (a[0] + a[2]) + (a[4] + a[6])
        hi = (a[1] + a[3]) + (a[5] + a[7])
        return lo * inv_h, hi * inv_h

    def group_body(g, _):
        grp_row = base_row + g * IC
        pltpu.sync_copy(x_hbm.at[pl.ds(grp_row, IC)], idx_v)
        for b in range(NBUF):
            fire(b, b)

        def ring_body(rr, _):
            for b in range(NBUF):
                r = rr * NBUF + b
                drain(r, b)

                @pl.when(rr < IC // NBUF - 1)
                def _():
                    fire(r + NBUF, b)

                lo, hi = accumulate(b)
                out_v[r, pl.ds(0, 16)] = lo
                out_v[r, pl.ds(16, 16)] = hi
            return 0

        lax.fori_loop(0, IC // NBUF, ring_body, 0)
        pltpu.sync_copy(out_v, out_hbm.at[pl.ds(grp_row, IC)])
        return 0

    lax.fori_loop(0, NGRP, group_body, 0)


def kernel(x, W):
    w_lin = _tc_repack(W.T).reshape(VP, D)
    return _encode(x, w_lin)


# speedup vs baseline: 1.6924x; 1.0187x over previous
import functools

import jax
import jax.numpy as jnp
from jax import lax
from jax.experimental import pallas as pl
from jax.experimental.pallas import tpu as pltpu
from jax.experimental.pallas import tpu_sc as plsc

VOCAB = 1000000
D = 32
B = 16384
H = 200

CV = 16384
NBLK = -(-VOCAB // CV)
VP = NBLK * CV

NC = 2
NS = 16
NW = NC * NS
RPW = B // NW
IC = 32
NGRP = RPW // IC
G0 = 128
G1 = H - G0
U = 8
NACC = 4
NBUF = 4

_mesh = plsc.VectorSubcoreMesh(
    core_axis_name="c", subcore_axis_name="s", num_cores=NC, num_subcores=NS
)


def _repack_body(in_ref, out_ref):
    tt = jnp.swapaxes(in_ref[...], 0, 1)
    r3 = tt.reshape(CV // 4, 4, D)
    for a in range(4):
        out_ref[:, D * a:D * (a + 1)] = r3[:, a, :]


def _tc_repack(wt):
    return pl.pallas_call(
        _repack_body,
        grid=(NBLK,),
        in_specs=[pl.BlockSpec((D, CV), lambda c: (0, c))],
        out_specs=pl.BlockSpec((CV // 4, 128), lambda c: (c, 0)),
        out_shape=jax.ShapeDtypeStruct((VP // 4, 128), jnp.float32),
    )(wt)


@functools.partial(
    pl.kernel,
    out_type=jax.ShapeDtypeStruct((B, D), jnp.float32),
    mesh=_mesh,
    compiler_params=pltpu.CompilerParams(use_tc_tiling_on_sc=False),
    scratch_types=[
        pltpu.VMEM((IC, H), jnp.int32),
        pltpu.VMEM((NBUF, H, D), jnp.float32),
        pltpu.VMEM((IC, D), jnp.float32),
    ]
    + [pltpu.SemaphoreType.DMA for _ in range(NBUF)],
)
def _encode(x_hbm, w_hbm, out_hbm, idx_v, rows_v, out_v, *sems):
    wid = lax.axis_index("s") * NC + lax.axis_index("c")
    base_row = wid * RPW
    inv_h = jnp.float32(1.0 / H)

    def copies(r, b):
        row_idx = idx_v.at[r]
        cp0 = pltpu.make_async_copy(
            w_hbm.at[row_idx.at[pl.ds(0, G0)]],
            rows_v.at[b].at[pl.ds(0, G0)],
            sems[b],
        )
        cp1 = pltpu.make_async_copy(
            w_hbm.at[row_idx.at[pl.ds(G0, G1)]],
            rows_v.at[b].at[pl.ds(G0, G1)],
            sems[b],
        )
        return cp0, cp1

    def fire(r, b):
        for cp in copies(r, b):
            cp.start()

    def drain(r, b):
        for cp in copies(r, b):
            cp.wait()

    def accumulate(b):
        def acc_body(it, carry):
            a = list(carry)
            base = it * U
            for u in range(U):
                j = base + u
                k = u % NACC
                a[2 * k] = a[2 * k] + rows_v[b, j, pl.ds(0, 16)]
                a[2 * k + 1] = a[2 * k + 1] + rows_v[b, j, pl.ds(16, 16)]
            return tuple(a)

        zeros = tuple(jnp.zeros((16,), jnp.float32) for _ in range(2 * NACC))
        a = lax.fori_loop(0, H // U, acc_body, zeros)
        lo = (a[0] + a[2]) + (a[4] + a[6])
        hi = (a[1] + a[3]) + (a[5] + a[7])
        return lo * inv_h, hi * inv_h

    def group_body(g, _):
        grp_row = base_row + g * IC
        pltpu.sync_copy(x_hbm.at[pl.ds(grp_row, IC)], idx_v)
        for b in range(NBUF):
            fire(b, b)

        def ring_body(rr, _):
            for b in range(NBUF):
                r = rr * NBUF + b
                drain(r, b)

                @pl.when(rr < IC // NBUF - 1)
                def _():
                    fire(r + NBUF, b)

                lo, hi = accumulate(b)
                out_v[r, pl.ds(0, 16)] = lo
                out_v[r, pl.ds(16, 16)] = hi
            return 0

        lax.fori_loop(0, IC // NBUF, ring_body, 0)
        pltpu.sync_copy(out_v, out_hbm.at[pl.ds(grp_row, IC)])
        return 0

    lax.fori_loop(0, NGRP, group_body, 0)


def kernel(x, W):
    w_lin = _tc_repack(W.T).reshape(VP, D)
    return _encode(x, w_lin)
